# baseline (speedup 1.0000x reference)
import functools
import math

import jax
import jax.numpy as jnp
from jax import lax
from jax.experimental import pallas as pl
from jax.experimental.pallas import tpu as pltpu
from jax.experimental.pallas import tpu_sc as plsc

_D = 1024
_SCALE = math.sqrt(_D)


@functools.cache
def _build(B):
    info = plsc.get_sparse_core_info()
    NC, NS, L = info.num_cores, info.num_subcores, info.num_lanes
    NW = NC * NS
    b_per_w = B // NW
    CHUNK = 32
    n_chunks = b_per_w // CHUNK
    mesh = plsc.VectorSubcoreMesh(core_axis_name="c", subcore_axis_name="s")

    @functools.partial(
        pl.kernel,
        mesh=mesh,
        out_type=jax.ShapeDtypeStruct((B, _D), jnp.float32),
        scratch_types=[
            pltpu.VMEM((n_chunks, CHUNK), jnp.int32),
            pltpu.VMEM((CHUNK, _D), jnp.float32),
            pltpu.SemaphoreType.DMA,
        ],
    )
    def emb(x_hbm, table_hbm, out_hbm, idx_v, buf, sem):
        wid = lax.axis_index("s") * NC + lax.axis_index("c")
        base = wid * b_per_w
        pltpu.sync_copy(x_hbm.at[wid], idx_v)

        def chunk_body(g, _):
            pltpu.async_copy(table_hbm.at[idx_v.at[g]], buf, sem).wait()

            def row_body(r, _):
                for j in range(_D // L):
                    sl = pl.ds(j * L, L)
                    buf[r, sl] = buf[r, sl] * _SCALE
                return 0

            lax.fori_loop(0, CHUNK, row_body, 0)
            pltpu.sync_copy(buf, out_hbm.at[pl.ds(base + g * CHUNK, CHUNK)])
            return 0

        lax.fori_loop(0, n_chunks, chunk_body, 0)

    return emb, NW, n_chunks, CHUNK


def kernel(x, table):
    B = x.size
    emb, NW, n_chunks, CHUNK = _build(B)
    xf = x.reshape(NW, n_chunks, CHUNK)
    out = emb(xf, table)
    return out.reshape(x.shape + (_D,))


# speedup vs baseline: 1.5274x; 1.5274x over previous
import functools
import math

import jax
import jax.numpy as jnp
from jax import lax
from jax.experimental import pallas as pl
from jax.experimental.pallas import tpu as pltpu
from jax.experimental.pallas import tpu_sc as plsc

_D = 1024
_SCALE = math.sqrt(_D)


@functools.cache
def _build(B):
    info = plsc.get_sparse_core_info()
    NC, NS, L = info.num_cores, info.num_subcores, info.num_lanes
    NW = NC * NS
    b_per_w = B // NW
    CHUNK = 16
    n_chunks = b_per_w // CHUNK
    mesh = plsc.VectorSubcoreMesh(core_axis_name="c", subcore_axis_name="s")

    @functools.partial(
        pl.kernel,
        mesh=mesh,
        out_type=jax.ShapeDtypeStruct((B, _D), jnp.float32),
        scratch_types=[
            pltpu.VMEM((n_chunks, CHUNK), jnp.int32),
            pltpu.VMEM((CHUNK, _D), jnp.float32),
            pltpu.VMEM((CHUNK, _D), jnp.float32),
            pltpu.VMEM((CHUNK, _D), jnp.float32),
            pltpu.VMEM((CHUNK, _D), jnp.float32),
            pltpu.SemaphoreType.DMA,
            pltpu.SemaphoreType.DMA,
            pltpu.SemaphoreType.DMA,
            pltpu.SemaphoreType.DMA,
        ],
    )
    def emb(x_hbm, table_hbm, out_hbm, idx_v, in0, in1, out0, out1,
            sg0, sg1, ss0, ss1):
        wid = lax.axis_index("s") * NC + lax.axis_index("c")
        base = wid * b_per_w
        pltpu.sync_copy(x_hbm.at[wid], idx_v)
        ins, outs = (in0, in1), (out0, out1)
        sgs, sss = (sg0, sg1), (ss0, ss1)

        def g_copy(g, b):
            return pltpu.make_async_copy(table_hbm.at[idx_v.at[g]], ins[b],
                                         sgs[b])

        def s_copy(g, b):
            return pltpu.make_async_copy(
                outs[b], out_hbm.at[pl.ds(base + g * CHUNK, CHUNK)], sss[b])

        g_copy(0, 0).start()
        g_copy(1, 1).start()

        def pair_body(p, _):
            for b in range(2):
                g = 2 * p + b
                g_copy(g, b).wait()

                @pl.when(g >= 2)
                def _():
                    s_copy(g - 2, b).wait()

                def row_body(r, _):
                    for j in range(_D // L):
                        sl = pl.ds(j * L, L)
                        outs[b][r, sl] = ins[b][r, sl] * _SCALE
                    return 0

                lax.fori_loop(0, CHUNK, row_body, 0)
                s_copy(g, b).start()

                @pl.when(g + 2 < n_chunks)
                def _():
                    g_copy(g + 2, b).start()
            return 0

        lax.fori_loop(0, n_chunks // 2, pair_body, 0)
        for b in range(2):
            s_copy(n_chunks - 2 + b, b).wait()

    return emb, NW, n_chunks, CHUNK


def kernel(x, table):
    B = x.size
    emb, NW, n_chunks, CHUNK = _build(B)
    xf = x.reshape(NW, n_chunks, CHUNK)
    out = emb(xf, table)
    return out.reshape(x.shape + (_D,))


# speedup vs baseline: 1.6287x; 1.0663x over previous
import functools
import math

import jax
import jax.numpy as jnp
from jax import lax
from jax.experimental import pallas as pl
from jax.experimental.pallas import tpu as pltpu
from jax.experimental.pallas import tpu_sc as plsc

_D = 1024
_SCALE = math.sqrt(_D)


@functools.cache
def _build(B):
    info = plsc.get_sparse_core_info()
    NC, NS, L = info.num_cores, info.num_subcores, info.num_lanes
    NW = NC * NS
    b_per_w = B // NW
    CHUNK = 8
    NBUF = 4
    n_chunks = b_per_w // CHUNK
    mesh = plsc.VectorSubcoreMesh(core_axis_name="c", subcore_axis_name="s")

    @functools.partial(
        pl.kernel,
        mesh=mesh,
        out_type=jax.ShapeDtypeStruct((B, _D), jnp.float32),
        scratch_types=[
            pltpu.VMEM((n_chunks, CHUNK), jnp.int32),
        ] + [pltpu.VMEM((CHUNK, _D), jnp.float32)] * (2 * NBUF)
          + [pltpu.SemaphoreType.DMA] * (2 * NBUF),
    )
    def emb(x_hbm, table_hbm, out_hbm, idx_v, *rest):
        ins = rest[:NBUF]
        outs = rest[NBUF:2 * NBUF]
        sgs = rest[2 * NBUF:3 * NBUF]
        sss = rest[3 * NBUF:4 * NBUF]
        wid = lax.axis_index("s") * NC + lax.axis_index("c")
        base = wid * b_per_w
        pltpu.sync_copy(x_hbm.at[wid], idx_v)

        def g_copy(g, b):
            return pltpu.make_async_copy(table_hbm.at[idx_v.at[g]], ins[b],
                                         sgs[b])

        def s_copy(g, b):
            return pltpu.make_async_copy(
                outs[b], out_hbm.at[pl.ds(base + g * CHUNK, CHUNK)], sss[b])

        for b in range(NBUF):
            g_copy(b, b).start()

        def group_body(p, _):
            for b in range(NBUF):
                g = NBUF * p + b
                g_copy(g, b).wait()

                @pl.when(g >= NBUF)
                def _():
                    s_copy(g - NBUF, b).wait()

                def row_body(r, _):
                    for j in range(_D // L):
                        sl = pl.ds(j * L, L)
                        outs[b][r, sl] = ins[b][r, sl] * _SCALE
                    return 0

                lax.fori_loop(0, CHUNK, row_body, 0)
                s_copy(g, b).start()

                @pl.when(g + NBUF < n_chunks)
                def _():
                    g_copy(g + NBUF, b).start()
            return 0

        lax.fori_loop(0, n_chunks // NBUF, group_body, 0)
        for b in range(NBUF):
            s_copy(n_chunks - NBUF + b, b).wait()

    return emb, NW, n_chunks, CHUNK


def kernel(x, table):
    B = x.size
    emb, NW, n_chunks, CHUNK = _build(B)
    xf = x.reshape(NW, n_chunks, CHUNK)
    out = emb(xf, table)
    return out.reshape(x.shape + (_D,))
